# Initial kernel scaffold; baseline (speedup 1.0000x reference)
import jax
import jax.numpy as jnp
from jax.experimental import pallas as pl


def kernel(x, x_src, cg_edge_index, batch, batch_src, W1, b1, W2, b2):
    raise NotImplementedError("write your pallas kernel here")


# speedup vs baseline: 2.5528x; 2.5528x over previous
import functools

import jax
import jax.numpy as jnp
from jax import lax
from jax.experimental import pallas as pl
from jax.experimental.pallas import tpu as pltpu
from jax.experimental.pallas import tpu_sc as plsc

N_NODES = 10000
N_EDGES = 160000
F_X = 256
HIDDEN = 512

NC = 2
NS = 16
L = 16
NW = NC * NS

K = 64
G = K // L
NCH0 = 50
NCH1 = 110
EPW0 = NCH0 * K
EPW1 = NCH1 * K
EPW_MAX = max(EPW0, EPW1)
E_PAD = NS * (EPW0 + EPW1)
E_IDX = E_PAD + (EPW_MAX - min(EPW0, EPW1))
ACC_ROWS = 10240
DUMMY = N_NODES


def _pack_bf16(v):
  lo = lax.bitcast_convert_type(
      v[:, : HIDDEN // 2].astype(jnp.bfloat16), jnp.uint16
  ).astype(jnp.uint32)
  hi = lax.bitcast_convert_type(
      v[:, HIDDEN // 2 :].astype(jnp.bfloat16), jnp.uint16
  ).astype(jnp.uint32)
  return lax.bitcast_convert_type(lo | (hi << 16), jnp.float32)


def _proj_body(x_ref, xs_ref, w1a_ref, w1b_ref, b1_ref, p_ref, q_ref):
  p_ref[...] = _pack_bf16(
      jnp.dot(x_ref[...], w1a_ref[...], preferred_element_type=jnp.float32)
      + b1_ref[...]
  )
  q_ref[...] = _pack_bf16(
      jnp.dot(xs_ref[...], w1b_ref[...], preferred_element_type=jnp.float32)
  )


def _project(x, x_src, w1a, w1b, b1):
  blk = 1000
  grid = (N_NODES // blk,)
  return pl.pallas_call(
      _proj_body,
      grid=grid,
      in_specs=[
          pl.BlockSpec((blk, F_X), lambda i: (i, 0)),
          pl.BlockSpec((blk, F_X), lambda i: (i, 0)),
          pl.BlockSpec((F_X, HIDDEN), lambda i: (0, 0)),
          pl.BlockSpec((F_X, HIDDEN), lambda i: (0, 0)),
          pl.BlockSpec((1, HIDDEN), lambda i: (0, 0)),
      ],
      out_specs=[
          pl.BlockSpec((blk, HIDDEN // 2), lambda i: (i, 0)),
          pl.BlockSpec((blk, HIDDEN // 2), lambda i: (i, 0)),
      ],
      out_shape=[
          jax.ShapeDtypeStruct((N_NODES, HIDDEN // 2), jnp.float32),
          jax.ShapeDtypeStruct((N_NODES, HIDDEN // 2), jnp.float32),
      ],
  )(x, x_src, w1a, w1b, b1)


_mesh = plsc.VectorSubcoreMesh(core_axis_name="c", subcore_axis_name="s")


@functools.partial(
    pl.kernel,
    out_type=jax.ShapeDtypeStruct((NW, ACC_ROWS), jnp.float32),
    mesh=_mesh,
    compiler_params=pltpu.CompilerParams(needs_layout_passes=False),
    scratch_types=[
        pltpu.VMEM((EPW_MAX,), jnp.int32),
        pltpu.VMEM((EPW_MAX,), jnp.int32),
        pltpu.VMEM((EPW_MAX,), jnp.int32),
        pltpu.VMEM((K, HIDDEN // 2), jnp.float32),
        pltpu.VMEM((K, HIDDEN // 2), jnp.float32),
        pltpu.VMEM((K, HIDDEN // 2), jnp.float32),
        pltpu.VMEM((K, HIDDEN // 2), jnp.float32),
        pltpu.VMEM((K * L,), jnp.float32),
        pltpu.VMEM((HIDDEN,), jnp.float32),
        pltpu.VMEM((L,), jnp.float32),
        pltpu.VMEM((ACC_ROWS,), jnp.float32),
        pltpu.SemaphoreType.DMA,
        pltpu.SemaphoreType.DMA,
        pltpu.SemaphoreType.DMA,
        pltpu.SemaphoreType.DMA,
    ],
)
def _edge_kernel(p_hbm, q_hbm, w2_hbm, b2_hbm, gd_hbm, gs_hbm, sd_hbm, out_hbm,
                 gd_all, gs_all, sd_all, rp0, rq0, rp1, rq1, acc_v, w2_v, b2_v,
                 accum_v, sp0, sq0, sp1, sq1):
  c = lax.axis_index("c")
  s = lax.axis_index("s")
  wid = s * NC + c

  zeros16 = jnp.zeros((L,), jnp.float32)

  def _zero_body(i, _):
    accum_v[pl.ds(i * L, L)] = zeros16
    return 0

  lax.fori_loop(0, ACC_ROWS // L, _zero_body, 0)

  base = jnp.where(c == 0, s * EPW0, NS * EPW0 + s * EPW1)
  nch = jnp.where(c == 0, NCH0, NCH1)
  pltpu.sync_copy(gd_hbm.at[pl.ds(base, EPW_MAX)], gd_all)
  pltpu.sync_copy(gs_hbm.at[pl.ds(base, EPW_MAX)], gs_all)
  pltpu.sync_copy(sd_hbm.at[pl.ds(base, EPW_MAX)], sd_all)
  pltpu.sync_copy(w2_hbm, w2_v)
  pltpu.sync_copy(b2_hbm, b2_v)
  w2s = [w2_v[pl.ds(i * L, L)] for i in range(HIDDEN // L)]
  acc0 = b2_v[...]
  lanes = lax.iota(jnp.int32, L)

  def _issue(jc, rp, rq, sp, sq):
    c0 = jnp.minimum(jc, nch - 1) * K
    pltpu.async_copy(p_hbm.at[gd_all.at[pl.ds(c0, K)]], rp, sp)
    pltpu.async_copy(q_hbm.at[gs_all.at[pl.ds(c0, K)]], rq, sq)

  def _wait(rp, rq, sp, sq):
    pltpu.make_async_copy(p_hbm.at[gd_all.at[pl.ds(0, K)]], rp, sp).wait()
    pltpu.make_async_copy(q_hbm.at[gs_all.at[pl.ds(0, K)]], rq, sq).wait()

  def _compute(jc, rp, rq):
    def _edge_body(e, _):
      acc = acc0
      for i in range(HIDDEN // (2 * L)):
        vp = plsc.bitcast(rp[e, pl.ds(i * L, L)], jnp.bfloat16)
        vq = plsc.bitcast(rq[e, pl.ds(i * L, L)], jnp.bfloat16)
        r = jnp.maximum(vp + vq, jnp.bfloat16(0.0))
        re, ro = plsc.unpack(r, format=plsc.PackFormat.INTERLEAVED)
        acc = acc + re * w2s[2 * i] + ro * w2s[2 * i + 1]
      acc_v[pl.ds(e * L, L)] = acc
      return 0

    lax.fori_loop(0, K, _edge_body, 0)

    def _group_body(g, _):
      base_idx = (g * L + lanes) * L
      tot = jnp.zeros((L,), jnp.float32)
      for col in range(L):
        tot = tot + plsc.load_gather(acc_v, [base_idx + col])
      sd16 = sd_all[pl.ds(jc * K + g * L, L)]
      plsc.addupdate_scatter(accum_v, [sd16], tot)
      return 0

    lax.fori_loop(0, G, _group_body, 0)

  _issue(0, rp0, rq0, sp0, sq0)

  def _chunk_body(j2, _):
    j = j2 * 2
    _issue(j + 1, rp1, rq1, sp1, sq1)
    _wait(rp0, rq0, sp0, sq0)
    _compute(j, rp0, rq0)
    _issue(j + 2, rp0, rq0, sp0, sq0)
    _wait(rp1, rq1, sp1, sq1)
    _compute(j + 1, rp1, rq1)
    return 0

  lax.fori_loop(0, nch // 2, _chunk_body, 0)
  _wait(rp0, rq0, sp0, sq0)

  pltpu.sync_copy(accum_v, out_hbm.at[wid])


def _combine_body(part_ref, o_ref):
  t = jnp.sum(part_ref[...], axis=0)
  o_ref[...] = jax.nn.sigmoid(t[:N_NODES])[:, None]


def _combine(partials):
  return pl.pallas_call(
      _combine_body,
      out_shape=jax.ShapeDtypeStruct((N_NODES, 1), jnp.float32),
  )(partials)


def kernel(x, x_src, cg_edge_index, batch, batch_src, W1, b1, W2, b2):
  del batch, batch_src
  src = cg_edge_index[0].astype(jnp.int32)
  dest = cg_edge_index[1].astype(jnp.int32)

  pad = E_IDX - N_EDGES
  gd = jnp.concatenate([dest, jnp.zeros((pad,), jnp.int32)])
  gs = jnp.concatenate([src, jnp.zeros((pad,), jnp.int32)])
  sd = jnp.concatenate([dest, jnp.full((pad,), DUMMY, jnp.int32)])

  w1a = W1[:F_X]
  w1b = W1[F_X:]
  b1r = b1.reshape(1, HIDDEN)
  b2v = jnp.full((L,), b2[0] / L, jnp.float32)
  p32, q32 = _project(x, x_src, w1a, w1b, b1r)

  w2_perm = jnp.stack(
      [W2[: HIDDEN // 2].reshape(L, L), W2[HIDDEN // 2 :].reshape(L, L)],
      axis=1,
  ).reshape(HIDDEN)
  partials = _edge_kernel(p32, q32, w2_perm, b2v, gd, gs, sd)
  out = _combine(partials)
  return out


# speedup vs baseline: 3.2439x; 1.2707x over previous
import functools

import jax
import jax.numpy as jnp
from jax import lax
from jax.experimental import pallas as pl
from jax.experimental.pallas import tpu as pltpu
from jax.experimental.pallas import tpu_sc as plsc

N_NODES = 10000
N_EDGES = 160000
F_X = 256
HIDDEN = 512

NC = 2
NS = 16
L = 16
NW = NC * NS

K = 64
G = K // L
NCH0 = 92
NCH1 = 68
EPW0 = NCH0 * K
EPW1 = NCH1 * K
EPW_MAX = max(EPW0, EPW1)
E_PAD = NS * (EPW0 + EPW1)
E_IDX = E_PAD + (EPW_MAX - min(EPW0, EPW1))
ACC_ROWS = 10240
DUMMY = N_NODES


def _pack_bf16(v):
  lo = lax.bitcast_convert_type(
      v[:, : HIDDEN // 2].astype(jnp.bfloat16), jnp.uint16
  ).astype(jnp.uint32)
  hi = lax.bitcast_convert_type(
      v[:, HIDDEN // 2 :].astype(jnp.bfloat16), jnp.uint16
  ).astype(jnp.uint32)
  return lax.bitcast_convert_type(lo | (hi << 16), jnp.float32)


def _proj_body(x_ref, xs_ref, w1a_ref, w1b_ref, b1_ref, p_ref, q_ref):
  p_ref[...] = _pack_bf16(
      jnp.dot(x_ref[...], w1a_ref[...], preferred_element_type=jnp.float32)
      + b1_ref[...]
  )
  q_ref[...] = _pack_bf16(
      jnp.dot(xs_ref[...], w1b_ref[...], preferred_element_type=jnp.float32)
  )


def _project(x, x_src, w1a, w1b, b1):
  blk = 1000
  grid = (N_NODES // blk,)
  return pl.pallas_call(
      _proj_body,
      grid=grid,
      in_specs=[
          pl.BlockSpec((blk, F_X), lambda i: (i, 0)),
          pl.BlockSpec((blk, F_X), lambda i: (i, 0)),
          pl.BlockSpec((F_X, HIDDEN), lambda i: (0, 0)),
          pl.BlockSpec((F_X, HIDDEN), lambda i: (0, 0)),
          pl.BlockSpec((1, HIDDEN), lambda i: (0, 0)),
      ],
      out_specs=[
          pl.BlockSpec((blk, HIDDEN // 2), lambda i: (i, 0)),
          pl.BlockSpec((blk, HIDDEN // 2), lambda i: (i, 0)),
      ],
      out_shape=[
          jax.ShapeDtypeStruct((N_NODES, HIDDEN // 2), jnp.float32),
          jax.ShapeDtypeStruct((N_NODES, HIDDEN // 2), jnp.float32),
      ],
  )(x, x_src, w1a, w1b, b1)


_mesh = plsc.VectorSubcoreMesh(core_axis_name="c", subcore_axis_name="s")


@functools.partial(
    pl.kernel,
    out_type=jax.ShapeDtypeStruct((NW, ACC_ROWS), jnp.float32),
    mesh=_mesh,
    compiler_params=pltpu.CompilerParams(needs_layout_passes=False),
    scratch_types=[
        pltpu.VMEM((EPW_MAX,), jnp.int32),
        pltpu.VMEM((EPW_MAX,), jnp.int32),
        pltpu.VMEM((EPW_MAX,), jnp.int32),
        pltpu.VMEM((K, HIDDEN // 2), jnp.float32),
        pltpu.VMEM((K, HIDDEN // 2), jnp.float32),
        pltpu.VMEM((K, HIDDEN // 2), jnp.float32),
        pltpu.VMEM((K, HIDDEN // 2), jnp.float32),
        pltpu.VMEM((K * L,), jnp.float32),
        pltpu.VMEM((HIDDEN,), jnp.float32),
        pltpu.VMEM((L,), jnp.float32),
        pltpu.VMEM((ACC_ROWS,), jnp.float32),
        pltpu.SemaphoreType.DMA,
        pltpu.SemaphoreType.DMA,
        pltpu.SemaphoreType.DMA,
        pltpu.SemaphoreType.DMA,
    ],
)
def _edge_kernel(p_hbm, q_hbm, w2_hbm, b2_hbm, gd_hbm, gs_hbm, sd_hbm, out_hbm,
                 gd_all, gs_all, sd_all, rp0, rq0, rp1, rq1, acc_v, w2_v, b2_v,
                 accum_v, sp0, sq0, sp1, sq1):
  c = lax.axis_index("c")
  s = lax.axis_index("s")
  wid = s * NC + c

  zeros16 = jnp.zeros((L,), jnp.float32)

  def _zero_body(i, _):
    accum_v[pl.ds(i * L, L)] = zeros16
    return 0

  lax.fori_loop(0, ACC_ROWS // L, _zero_body, 0)

  base = jnp.where(c == 0, s * EPW0, NS * EPW0 + s * EPW1)
  nch = jnp.where(c == 0, NCH0, NCH1)
  pltpu.sync_copy(gd_hbm.at[pl.ds(base, EPW_MAX)], gd_all)
  pltpu.sync_copy(gs_hbm.at[pl.ds(base, EPW_MAX)], gs_all)
  pltpu.sync_copy(sd_hbm.at[pl.ds(base, EPW_MAX)], sd_all)
  pltpu.sync_copy(w2_hbm, w2_v)
  pltpu.sync_copy(b2_hbm, b2_v)
  w2s = [w2_v[pl.ds(i * L, L)] for i in range(HIDDEN // L)]
  acc0 = b2_v[...]
  lanes = lax.iota(jnp.int32, L)

  def _issue(jc, rp, rq, sp, sq):
    c0 = jnp.minimum(jc, nch - 1) * K
    pltpu.async_copy(p_hbm.at[gd_all.at[pl.ds(c0, K)]], rp, sp)
    pltpu.async_copy(q_hbm.at[gs_all.at[pl.ds(c0, K)]], rq, sq)

  def _wait(rp, rq, sp, sq):
    pltpu.make_async_copy(p_hbm.at[gd_all.at[pl.ds(0, K)]], rp, sp).wait()
    pltpu.make_async_copy(q_hbm.at[gs_all.at[pl.ds(0, K)]], rq, sq).wait()

  def _compute(jc, rp, rq):
    def _edge_body(e, _):
      acc = acc0
      for i in range(HIDDEN // (2 * L)):
        vp = plsc.bitcast(rp[e, pl.ds(i * L, L)], jnp.bfloat16)
        vq = plsc.bitcast(rq[e, pl.ds(i * L, L)], jnp.bfloat16)
        r = jnp.maximum(vp + vq, jnp.bfloat16(0.0))
        re, ro = plsc.unpack(r, format=plsc.PackFormat.INTERLEAVED)
        acc = acc + re * w2s[2 * i] + ro * w2s[2 * i + 1]
      acc_v[pl.ds(e * L, L)] = acc
      return 0

    lax.fori_loop(0, K, _edge_body, 0)

    def _group_body(g, _):
      base_idx = (g * L + lanes) * L
      tot = jnp.zeros((L,), jnp.float32)
      for col in range(L):
        tot = tot + plsc.load_gather(acc_v, [base_idx + col])
      sd16 = sd_all[pl.ds(jc * K + g * L, L)]
      plsc.addupdate_scatter(accum_v, [sd16], tot)
      return 0

    lax.fori_loop(0, G, _group_body, 0)

  _issue(0, rp0, rq0, sp0, sq0)

  def _chunk_body(j2, _):
    j = j2 * 2
    _issue(j + 1, rp1, rq1, sp1, sq1)
    _wait(rp0, rq0, sp0, sq0)
    _compute(j, rp0, rq0)
    _issue(j + 2, rp0, rq0, sp0, sq0)
    _wait(rp1, rq1, sp1, sq1)
    _compute(j + 1, rp1, rq1)
    return 0

  lax.fori_loop(0, nch // 2, _chunk_body, 0)
  _wait(rp0, rq0, sp0, sq0)

  pltpu.sync_copy(accum_v, out_hbm.at[wid])


def _combine_body(part_ref, o_ref):
  t = jnp.sum(part_ref[...], axis=0)
  o_ref[...] = jax.nn.sigmoid(t[:N_NODES])[:, None]


def _combine(partials):
  return pl.pallas_call(
      _combine_body,
      out_shape=jax.ShapeDtypeStruct((N_NODES, 1), jnp.float32),
  )(partials)


def kernel(x, x_src, cg_edge_index, batch, batch_src, W1, b1, W2, b2):
  del batch, batch_src
  src = cg_edge_index[0].astype(jnp.int32)
  dest = cg_edge_index[1].astype(jnp.int32)

  pad = E_IDX - N_EDGES
  gd = jnp.concatenate([dest, jnp.zeros((pad,), jnp.int32)])
  gs = jnp.concatenate([src, jnp.zeros((pad,), jnp.int32)])
  sd = jnp.concatenate([dest, jnp.full((pad,), DUMMY, jnp.int32)])

  w1a = W1[:F_X]
  w1b = W1[F_X:]
  b1r = b1.reshape(1, HIDDEN)
  b2v = jnp.full((L,), b2[0] / L, jnp.float32)
  p32, q32 = _project(x, x_src, w1a, w1b, b1r)

  w2_perm = jnp.stack(
      [W2[: HIDDEN // 2].reshape(L, L), W2[HIDDEN // 2 :].reshape(L, L)],
      axis=1,
  ).reshape(HIDDEN)
  partials = _edge_kernel(p32, q32, w2_perm, b2v, gd, gs, sd)
  out = _combine(partials)
  return out
